# baseline (speedup 1.0000x reference)
import functools
import jax
import jax.numpy as jnp
from jax import lax
from jax.experimental import pallas as pl
from jax.experimental.pallas import tpu as pltpu
from jax.experimental.pallas import tpu_sc as plsc

B = 1024
L = 200
D_WORD = 300
D_POS = 5
D_OUT = D_WORD + 2 * D_POS
D_PAD = 304
NPOS = 400

N = B * L
CH = 64
NC = 2
NS = 16
NW = NC * NS
N_PER_W = N // NW
CHUNKS = N_PER_W // CH
G = 16


def _assemble(i, wbuf_v, obuf_v, hidx_v, tidx_v, p1_v, p2_v, iota):
  zero = jnp.full((16,), 0, jnp.int32)

  @pl.loop(0, CH // G)
  def group(g):
    r0 = g * G
    rows = iota + r0
    for rr in range(G):
      r = r0 + rr
      tok = jnp.full((16,), r, jnp.int32)
      for k in range(D_PAD // G):
        vals = wbuf_v[r, pl.ds(k * G, G)]
        plsc.store_scatter(obuf_v, [iota + k * G, zero, zero, zero, tok], vals)
    hi = hidx_v[i, pl.ds(r0, G)] * D_POS
    ti = tidx_v[i, pl.ds(r0, G)] * D_POS
    for c in range(D_POS):
      vals = plsc.load_gather(p1_v, [hi + c])
      plsc.store_scatter(obuf_v, [jnp.full((16,), D_WORD + c, jnp.int32), zero, zero, zero, rows], vals)
    for c in range(D_POS):
      vals = plsc.load_gather(p2_v, [ti + c])
      plsc.store_scatter(obuf_v, [jnp.full((16,), D_WORD + D_POS + c, jnp.int32), zero, zero, zero, rows], vals)


def _body(word_hbm, posh_hbm, post_hbm, wtab_hbm, p1_hbm, p2_hbm, out_hbm,
          widx_v, hidx_v, tidx_v, p1_v, p2_v,
          wbuf0, wbuf1, obuf0, obuf1, sem_g0, sem_g1, sem_w0, sem_w1):
  wid = lax.axis_index("s") * NC + lax.axis_index("c")
  crow = wid * CHUNKS

  pltpu.sync_copy(word_hbm.at[wid], widx_v)
  pltpu.sync_copy(posh_hbm.at[wid], hidx_v)
  pltpu.sync_copy(post_hbm.at[wid], tidx_v)
  pltpu.sync_copy(p1_hbm, p1_v)
  pltpu.sync_copy(p2_hbm, p2_v)

  iota = lax.iota(jnp.int32, 16)

  def gather(i, wbuf, sem):
    return pltpu.async_copy(wtab_hbm.at[widx_v.at[i]], wbuf, sem)

  def out_dst(i):
    c0 = crow + i
    lt = lax.shift_right_logical(c0, 7)
    bt = lax.bitwise_and(lax.shift_right_logical(c0, 4), 7)
    ls = lax.bitwise_and(lax.shift_right_logical(c0, 1), 7)
    h = lax.bitwise_and(c0, 1)
    return out_hbm.at[:, pl.ds(lt, 1), pl.ds(bt, 1), pl.ds(ls, 1), pl.ds(h * CH, CH)]

  def writeback(i, obuf, sem):
    return pltpu.async_copy(obuf, out_dst(i), sem)

  gather(0, wbuf0, sem_g0)

  @pl.loop(0, CHUNKS, step=2)
  def pair(i):
    gather(i + 1, wbuf1, sem_g1)
    pltpu.make_async_copy(wtab_hbm.at[widx_v.at[i]], wbuf0, sem_g0).wait()

    @pl.when(i > 0)
    def _():
      pltpu.make_async_copy(obuf0, out_dst(i - 2), sem_w0).wait()

    _assemble(i, wbuf0, obuf0, hidx_v, tidx_v, p1_v, p2_v, iota)
    writeback(i, obuf0, sem_w0)

    @pl.when(i + 2 < CHUNKS)
    def _():
      gather(i + 2, wbuf0, sem_g0)
    pltpu.make_async_copy(wtab_hbm.at[widx_v.at[i + 1]], wbuf1, sem_g1).wait()

    @pl.when(i > 0)
    def _():
      pltpu.make_async_copy(obuf1, out_dst(i - 1), sem_w1).wait()

    _assemble(i + 1, wbuf1, obuf1, hidx_v, tidx_v, p1_v, p2_v, iota)
    writeback(i + 1, obuf1, sem_w1)

  pltpu.make_async_copy(obuf0, out_dst(CHUNKS - 2), sem_w0).wait()
  pltpu.make_async_copy(obuf1, out_dst(CHUNKS - 1), sem_w1).wait()


V_PAD = 100096
CTILES = D_PAD // 8
WTILES = V_PAD // 128


def _tbody(wt4_hbm, out_hbm, slab_v, row_v, sem_in, sem_w):
  wid = lax.axis_index("s") * NC + lax.axis_index("c")
  nloop = (WTILES + NW - 1) // NW
  iota = lax.iota(jnp.int32, 16)

  @pl.loop(0, nloop)
  def step(j):
    g = j * NW + wid

    @pl.when(g < WTILES)
    def _():
      pltpu.async_copy(wt4_hbm.at[:, pl.ds(g, 1)], slab_v, sem_in).wait()

      @pl.loop(0, CTILES)
      def ca(a):
        for cs in range(8):
          col = jnp.full((16,), 8 * a + cs, jnp.int32)
          for wg in range(8):
            vals = slab_v[a, 0, cs, pl.ds(wg * 16, 16)]
            plsc.store_scatter(row_v, [iota + wg * 16, col], vals)

      pltpu.async_copy(row_v, out_hbm.at[pl.ds(g * 128, 128)], sem_w).wait()


_tmesh = plsc.VectorSubcoreMesh(core_axis_name="c", subcore_axis_name="s")

_transpose_call = pl.kernel(
    _tbody,
    out_type=jax.ShapeDtypeStruct((V_PAD, D_PAD), jnp.float32),
    mesh=_tmesh,
    scratch_types=[
        pltpu.VMEM((CTILES, 1, 8, 128), jnp.float32),
        pltpu.VMEM((128, D_PAD), jnp.float32),
        pltpu.SemaphoreType.DMA,
        pltpu.SemaphoreType.DMA,
    ],
    compiler_params=pltpu.CompilerParams(
        use_tc_tiling_on_sc=False, needs_layout_passes=False),
)


_mesh = plsc.VectorSubcoreMesh(core_axis_name="c", subcore_axis_name="s")

_sc_call = pl.kernel(
    _body,
    out_type=jax.ShapeDtypeStruct((D_OUT, L // 8, B // 128, 8, 128), jnp.float32),
    mesh=_mesh,
    scratch_types=[
        pltpu.VMEM((CHUNKS, CH), jnp.int32),
        pltpu.VMEM((CHUNKS, CH), jnp.int32),
        pltpu.VMEM((CHUNKS, CH), jnp.int32),
        pltpu.VMEM((NPOS * D_POS,), jnp.float32),
        pltpu.VMEM((NPOS * D_POS,), jnp.float32),
        pltpu.VMEM((CH, D_PAD), jnp.float32),
        pltpu.VMEM((CH, D_PAD), jnp.float32),
        pltpu.VMEM((D_OUT, 1, 1, 1, CH), jnp.float32),
        pltpu.VMEM((D_OUT, 1, 1, 1, CH), jnp.float32),
        pltpu.SemaphoreType.DMA,
        pltpu.SemaphoreType.DMA,
        pltpu.SemaphoreType.DMA,
        pltpu.SemaphoreType.DMA,
    ],
    compiler_params=pltpu.CompilerParams(
        use_tc_tiling_on_sc=False, needs_layout_passes=False),
)


def _tile_view(a):
  return (a.T.reshape(L // 8, 8, B // 128, 128).transpose(0, 2, 1, 3)
          .reshape(NW, CHUNKS, CH))


@jax.jit
def kernel(word, posh, post, word_table, pos1_table, pos2_table):
  w = _tile_view(word.astype(jnp.int32))
  h = _tile_view(posh.astype(jnp.int32))
  t = _tile_view(post.astype(jnp.int32))
  p1 = pos1_table.reshape(NPOS * D_POS)
  p2 = pos2_table.reshape(NPOS * D_POS)
  wtP = jnp.pad(word_table, ((0, V_PAD - word_table.shape[0]), (0, D_PAD - D_WORD)))
  wt4 = wtP.T.reshape(CTILES, 8, WTILES, 128).transpose(0, 2, 1, 3)
  wtRM = _transpose_call(wt4)
  out5 = _sc_call(w, h, t, wtRM, p1, p2)
  out = (out5.transpose(1, 3, 2, 4, 0)
         .reshape(L, B, D_OUT).transpose(1, 0, 2))
  return out


# speedup vs baseline: 1.4577x; 1.4577x over previous
import functools
import jax
import jax.numpy as jnp
from jax import lax
from jax.experimental import pallas as pl
from jax.experimental.pallas import tpu as pltpu
from jax.experimental.pallas import tpu_sc as plsc

B = 1024
L = 200
D_WORD = 300
D_POS = 5
D_OUT = D_WORD + 2 * D_POS
D_PAD = 304
NPOS = 400

N = B * L
CH = 64
NC = 2
NS = 16
NW = NC * NS
N_PER_W = N // NW
CHUNKS = N_PER_W // CH
G = 16


def _assemble(i, wbuf_v, obuf_v, hidx_v, tidx_v, p1_v, p2_v, iota):
  @pl.loop(0, CH // G)
  def group(g):
    r0 = g * G
    rows = iota + r0
    for rr in range(G):
      r = r0 + rr
      rg = r // 8
      rs = r % 8
      for k in range(D_PAD // G):
        obuf_v[rg, k // 8, rs, pl.ds((k % 8) * G, G)] = wbuf_v[r, pl.ds(k * G, G)]
    hi = hidx_v[i, pl.ds(r0, G)] * D_POS
    ti = tidx_v[i, pl.ds(r0, G)] * D_POS
    rg16 = lax.shift_right_logical(rows, 3)
    rs16 = lax.bitwise_and(rows, 7)
    two = jnp.full((16,), 2, jnp.int32)
    for c in range(D_POS):
      vals = plsc.load_gather(p1_v, [hi + c])
      plsc.store_scatter(obuf_v, [rg16, two, rs16, jnp.full((16,), D_WORD - 256 + c, jnp.int32)], vals)
    for c in range(D_POS):
      vals = plsc.load_gather(p2_v, [ti + c])
      plsc.store_scatter(obuf_v, [rg16, two, rs16, jnp.full((16,), D_WORD - 256 + D_POS + c, jnp.int32)], vals)


def _body(word_hbm, posh_hbm, post_hbm, wtab_hbm, p1_hbm, p2_hbm, out_hbm,
          widx_v, hidx_v, tidx_v, p1_v, p2_v,
          wbuf0, wbuf1, obuf0, obuf1, sem_g0, sem_g1, sem_w0, sem_w1):
  wid = lax.axis_index("s") * NC + lax.axis_index("c")
  crow = wid * CHUNKS

  pltpu.sync_copy(word_hbm.at[wid], widx_v)
  pltpu.sync_copy(posh_hbm.at[wid], hidx_v)
  pltpu.sync_copy(post_hbm.at[wid], tidx_v)
  pltpu.sync_copy(p1_hbm, p1_v)
  pltpu.sync_copy(p2_hbm, p2_v)

  iota = lax.iota(jnp.int32, 16)

  def gather(i, wbuf, sem):
    return pltpu.async_copy(wtab_hbm.at[widx_v.at[i]], wbuf, sem)

  def writeback(i, obuf, sem):
    return pltpu.async_copy(obuf, out_hbm.at[pl.ds((crow + i) * (CH // 8), CH // 8)], sem)

  gather(0, wbuf0, sem_g0)

  @pl.loop(0, CHUNKS, step=2)
  def pair(i):
    gather(i + 1, wbuf1, sem_g1)
    pltpu.make_async_copy(wtab_hbm.at[widx_v.at[i]], wbuf0, sem_g0).wait()

    @pl.when(i > 0)
    def _():
      pltpu.make_async_copy(obuf0, out_hbm.at[pl.ds((crow + i - 2) * (CH // 8), CH // 8)], sem_w0).wait()

    _assemble(i, wbuf0, obuf0, hidx_v, tidx_v, p1_v, p2_v, iota)
    writeback(i, obuf0, sem_w0)

    @pl.when(i + 2 < CHUNKS)
    def _():
      gather(i + 2, wbuf0, sem_g0)
    pltpu.make_async_copy(wtab_hbm.at[widx_v.at[i + 1]], wbuf1, sem_g1).wait()

    @pl.when(i > 0)
    def _():
      pltpu.make_async_copy(obuf1, out_hbm.at[pl.ds((crow + i - 1) * (CH // 8), CH // 8)], sem_w1).wait()

    _assemble(i + 1, wbuf1, obuf1, hidx_v, tidx_v, p1_v, p2_v, iota)
    writeback(i + 1, obuf1, sem_w1)

  pltpu.make_async_copy(obuf0, out_hbm.at[pl.ds((crow + CHUNKS - 2) * (CH // 8), CH // 8)], sem_w0).wait()
  pltpu.make_async_copy(obuf1, out_hbm.at[pl.ds((crow + CHUNKS - 1) * (CH // 8), CH // 8)], sem_w1).wait()


V_PAD = 100096
CTILES = D_PAD // 8
WTILES = V_PAD // 128


def _tbody(wt4_hbm, out_hbm, slab_v, row_v, sem_in, sem_w):
  wid = lax.axis_index("s") * NC + lax.axis_index("c")
  nloop = (WTILES + NW - 1) // NW
  iota = lax.iota(jnp.int32, 16)

  @pl.loop(0, nloop)
  def step(j):
    g = j * NW + wid

    @pl.when(g < WTILES)
    def _():
      pltpu.async_copy(wt4_hbm.at[:, pl.ds(g, 1)], slab_v, sem_in).wait()

      @pl.loop(0, CTILES)
      def ca(a):
        for cs in range(8):
          col = jnp.full((16,), 8 * a + cs, jnp.int32)
          for wg in range(8):
            vals = slab_v[a, 0, cs, pl.ds(wg * 16, 16)]
            plsc.store_scatter(row_v, [iota + wg * 16, col], vals)

      pltpu.async_copy(row_v, out_hbm.at[pl.ds(g * 128, 128)], sem_w).wait()


_tmesh = plsc.VectorSubcoreMesh(core_axis_name="c", subcore_axis_name="s")

_transpose_call = pl.kernel(
    _tbody,
    out_type=jax.ShapeDtypeStruct((V_PAD, D_PAD), jnp.float32),
    mesh=_tmesh,
    scratch_types=[
        pltpu.VMEM((CTILES, 1, 8, 128), jnp.float32),
        pltpu.VMEM((128, D_PAD), jnp.float32),
        pltpu.SemaphoreType.DMA,
        pltpu.SemaphoreType.DMA,
    ],
    compiler_params=pltpu.CompilerParams(
        use_tc_tiling_on_sc=False, needs_layout_passes=False),
)


_mesh = plsc.VectorSubcoreMesh(core_axis_name="c", subcore_axis_name="s")

_sc_call = pl.kernel(
    _body,
    out_type=jax.ShapeDtypeStruct((N // 8, 3, 8, 128), jnp.float32),
    mesh=_mesh,
    scratch_types=[
        pltpu.VMEM((CHUNKS, CH), jnp.int32),
        pltpu.VMEM((CHUNKS, CH), jnp.int32),
        pltpu.VMEM((CHUNKS, CH), jnp.int32),
        pltpu.VMEM((NPOS * D_POS,), jnp.float32),
        pltpu.VMEM((NPOS * D_POS,), jnp.float32),
        pltpu.VMEM((CH, D_PAD), jnp.float32),
        pltpu.VMEM((CH, D_PAD), jnp.float32),
        pltpu.VMEM((CH // 8, 3, 8, 128), jnp.float32),
        pltpu.VMEM((CH // 8, 3, 8, 128), jnp.float32),
        pltpu.SemaphoreType.DMA,
        pltpu.SemaphoreType.DMA,
        pltpu.SemaphoreType.DMA,
        pltpu.SemaphoreType.DMA,
    ],
    compiler_params=pltpu.CompilerParams(
        use_tc_tiling_on_sc=False, needs_layout_passes=False),
)


@jax.jit
def kernel(word, posh, post, word_table, pos1_table, pos2_table):
  w = word.reshape(NW, CHUNKS, CH).astype(jnp.int32)
  h = posh.reshape(NW, CHUNKS, CH).astype(jnp.int32)
  t = post.reshape(NW, CHUNKS, CH).astype(jnp.int32)
  p1 = pos1_table.reshape(NPOS * D_POS)
  p2 = pos2_table.reshape(NPOS * D_POS)
  wtP = jnp.pad(word_table, ((0, V_PAD - word_table.shape[0]), (0, D_PAD - D_WORD)))
  wt4 = wtP.T.reshape(CTILES, 8, WTILES, 128).transpose(0, 2, 1, 3)
  wtRM = _transpose_call(wt4)
  out4 = _sc_call(w, h, t, wtRM, p1, p2)
  out = out4.transpose(0, 2, 1, 3).reshape(N, 384)[:, :D_OUT]
  return out.reshape(B, L, D_OUT)


# speedup vs baseline: 1.5374x; 1.0546x over previous
import functools
import jax
import jax.numpy as jnp
from jax import lax
from jax.experimental import pallas as pl
from jax.experimental.pallas import tpu as pltpu
from jax.experimental.pallas import tpu_sc as plsc

B = 1024
L = 200
D_WORD = 300
D_POS = 5
D_OUT = D_WORD + 2 * D_POS
D_PAD = 304
NPOS = 400

N = B * L
CH = 64
NC = 2
NS = 16
NW = NC * NS
N_PER_W = N // NW
CHUNKS = N_PER_W // CH
G = 16


def _assemble(i, wbuf_v, obuf_v, hidx_v, tidx_v, p1_v, p2_v, iota):
  @pl.loop(0, CH // G)
  def group(g):
    r0 = g * G
    rows = iota + r0
    for rr in range(G):
      r = r0 + rr
      rg = r // 8
      rs = r % 8
      for k in range(D_PAD // G):
        obuf_v[rg, k // 8, rs, pl.ds((k % 8) * G, G)] = wbuf_v[r, pl.ds(k * G, G)]
    hi = hidx_v[i, pl.ds(r0, G)] * D_POS
    ti = tidx_v[i, pl.ds(r0, G)] * D_POS
    rg16 = lax.shift_right_logical(rows, 3)
    rs16 = lax.bitwise_and(rows, 7)
    two = jnp.full((16,), 2, jnp.int32)
    for c in range(D_POS):
      vals = plsc.load_gather(p1_v, [hi + c])
      plsc.store_scatter(obuf_v, [rg16, two, rs16, jnp.full((16,), D_WORD - 256 + c, jnp.int32)], vals)
    for c in range(D_POS):
      vals = plsc.load_gather(p2_v, [ti + c])
      plsc.store_scatter(obuf_v, [rg16, two, rs16, jnp.full((16,), D_WORD - 256 + D_POS + c, jnp.int32)], vals)


def _body(word_hbm, posh_hbm, post_hbm, wtab_hbm, p1_hbm, p2_hbm, out_hbm,
          widx_v, hidx_v, tidx_v, p1_v, p2_v,
          wbuf0, wbuf1, obuf0, obuf1, sem_g0, sem_g1, sem_w0, sem_w1):
  wid = lax.axis_index("s") * NC + lax.axis_index("c")
  crow = wid * CHUNKS

  pltpu.sync_copy(word_hbm.at[wid], widx_v)
  pltpu.sync_copy(posh_hbm.at[wid], hidx_v)
  pltpu.sync_copy(post_hbm.at[wid], tidx_v)
  pltpu.sync_copy(p1_hbm, p1_v)
  pltpu.sync_copy(p2_hbm, p2_v)

  iota = lax.iota(jnp.int32, 16)

  def gather(i, wbuf, sem):
    return pltpu.async_copy(wtab_hbm.at[widx_v.at[i]], wbuf, sem)

  def writeback(i, obuf, sem):
    return pltpu.async_copy(obuf, out_hbm.at[pl.ds((crow + i) * (CH // 8), CH // 8)], sem)

  gather(0, wbuf0, sem_g0)

  @pl.loop(0, CHUNKS, step=2)
  def pair(i):
    gather(i + 1, wbuf1, sem_g1)
    pltpu.make_async_copy(wtab_hbm.at[widx_v.at[i]], wbuf0, sem_g0).wait()

    @pl.when(i > 0)
    def _():
      pltpu.make_async_copy(obuf0, out_hbm.at[pl.ds((crow + i - 2) * (CH // 8), CH // 8)], sem_w0).wait()

    _assemble(i, wbuf0, obuf0, hidx_v, tidx_v, p1_v, p2_v, iota)
    writeback(i, obuf0, sem_w0)

    @pl.when(i + 2 < CHUNKS)
    def _():
      gather(i + 2, wbuf0, sem_g0)
    pltpu.make_async_copy(wtab_hbm.at[widx_v.at[i + 1]], wbuf1, sem_g1).wait()

    @pl.when(i > 0)
    def _():
      pltpu.make_async_copy(obuf1, out_hbm.at[pl.ds((crow + i - 1) * (CH // 8), CH // 8)], sem_w1).wait()

    _assemble(i + 1, wbuf1, obuf1, hidx_v, tidx_v, p1_v, p2_v, iota)
    writeback(i + 1, obuf1, sem_w1)

  pltpu.make_async_copy(obuf0, out_hbm.at[pl.ds((crow + CHUNKS - 2) * (CH // 8), CH // 8)], sem_w0).wait()
  pltpu.make_async_copy(obuf1, out_hbm.at[pl.ds((crow + CHUNKS - 1) * (CH // 8), CH // 8)], sem_w1).wait()


V_PAD = 100096
CTILES = D_PAD // 8
WTILES = V_PAD // 128


def _tbody(wt4_hbm, out_hbm, slab0, slab1, row_v, sem_i0, sem_i1, sem_w):
  wid = lax.axis_index("s") * NC + lax.axis_index("c")
  iota = lax.iota(jnp.int32, 16)

  def stage(j, slab, sem):
    return pltpu.async_copy(wt4_hbm.at[:, pl.ds(j * NW + wid, 1)], slab, sem)

  def halfstep(j, slab, sem):
    g = j * NW + wid

    @pl.when(g < WTILES)
    def _():
      pltpu.make_async_copy(wt4_hbm.at[:, pl.ds(g, 1)], slab, sem).wait()

      @pl.loop(0, CTILES)
      def ca(a):
        for cs in range(8):
          col = jnp.full((16,), 8 * a + cs, jnp.int32)
          for wg in range(8):
            vals = slab[a, 0, cs, pl.ds(wg * 16, 16)]
            plsc.store_scatter(row_v, [iota + wg * 16, col], vals)

      pltpu.async_copy(row_v, out_hbm.at[pl.ds(g * 128, 128)], sem_w).wait()

  stage(0, slab0, sem_i0)

  @pl.loop(0, 26, step=2)
  def pair(j):
    @pl.when((j + 1) * NW + wid < WTILES)
    def _():
      stage(j + 1, slab1, sem_i1)
    halfstep(j, slab0, sem_i0)

    @pl.when((j + 2) * NW + wid < WTILES)
    def _():
      stage(j + 2, slab0, sem_i0)
    halfstep(j + 1, slab1, sem_i1)


_tmesh = plsc.VectorSubcoreMesh(core_axis_name="c", subcore_axis_name="s")

_transpose_call = pl.kernel(
    _tbody,
    out_type=jax.ShapeDtypeStruct((V_PAD, D_PAD), jnp.float32),
    mesh=_tmesh,
    scratch_types=[
        pltpu.VMEM((CTILES, 1, 8, 128), jnp.float32),
        pltpu.VMEM((CTILES, 1, 8, 128), jnp.float32),
        pltpu.VMEM((128, D_PAD), jnp.float32),
        pltpu.SemaphoreType.DMA,
        pltpu.SemaphoreType.DMA,
        pltpu.SemaphoreType.DMA,
    ],
    compiler_params=pltpu.CompilerParams(
        use_tc_tiling_on_sc=False, needs_layout_passes=False),
)


_mesh = plsc.VectorSubcoreMesh(core_axis_name="c", subcore_axis_name="s")

_sc_call = pl.kernel(
    _body,
    out_type=jax.ShapeDtypeStruct((N // 8, 3, 8, 128), jnp.float32),
    mesh=_mesh,
    scratch_types=[
        pltpu.VMEM((CHUNKS, CH), jnp.int32),
        pltpu.VMEM((CHUNKS, CH), jnp.int32),
        pltpu.VMEM((CHUNKS, CH), jnp.int32),
        pltpu.VMEM((NPOS * D_POS,), jnp.float32),
        pltpu.VMEM((NPOS * D_POS,), jnp.float32),
        pltpu.VMEM((CH, D_PAD), jnp.float32),
        pltpu.VMEM((CH, D_PAD), jnp.float32),
        pltpu.VMEM((CH // 8, 3, 8, 128), jnp.float32),
        pltpu.VMEM((CH // 8, 3, 8, 128), jnp.float32),
        pltpu.SemaphoreType.DMA,
        pltpu.SemaphoreType.DMA,
        pltpu.SemaphoreType.DMA,
        pltpu.SemaphoreType.DMA,
    ],
    compiler_params=pltpu.CompilerParams(
        use_tc_tiling_on_sc=False, needs_layout_passes=False),
)


@jax.jit
def kernel(word, posh, post, word_table, pos1_table, pos2_table):
  w = word.reshape(NW, CHUNKS, CH).astype(jnp.int32)
  h = posh.reshape(NW, CHUNKS, CH).astype(jnp.int32)
  t = post.reshape(NW, CHUNKS, CH).astype(jnp.int32)
  p1 = pos1_table.reshape(NPOS * D_POS)
  p2 = pos2_table.reshape(NPOS * D_POS)
  wtP = jnp.pad(word_table, ((0, V_PAD - word_table.shape[0]), (0, D_PAD - D_WORD)))
  wt4 = wtP.T.reshape(CTILES, 8, WTILES, 128).transpose(0, 2, 1, 3)
  wtRM = _transpose_call(wt4)
  out4 = _sc_call(w, h, t, wtRM, p1, p2)
  out = out4.transpose(0, 2, 1, 3).reshape(N, 384)[:, :D_OUT]
  return out.reshape(B, L, D_OUT)


# speedup vs baseline: 1.5909x; 1.0348x over previous
import functools
import jax
import jax.numpy as jnp
from jax import lax
from jax.experimental import pallas as pl
from jax.experimental.pallas import tpu as pltpu
from jax.experimental.pallas import tpu_sc as plsc

B = 1024
L = 200
D_WORD = 300
D_POS = 5
D_OUT = D_WORD + 2 * D_POS
D_PAD = 304
NPOS = 400

N = B * L
CH = 64
NC = 2
NS = 16
NW = NC * NS
N_PER_W = N // NW
CHUNKS = N_PER_W // CH
G = 16


def _assemble(i, wbuf_v, obuf_v, hidx_v, tidx_v, p1_v, p2_v, iota):
  @pl.loop(0, CH // G)
  def group(g):
    r0 = g * G
    rows = iota + r0
    for rr in range(G):
      r = r0 + rr
      rg = r // 8
      rs = r % 8
      for k in range(D_PAD // G):
        obuf_v[rg, k // 8, rs, pl.ds((k % 8) * G, G)] = wbuf_v[r, pl.ds(k * G, G)]
    hi = hidx_v[i, pl.ds(r0, G)] * D_POS
    ti = tidx_v[i, pl.ds(r0, G)] * D_POS
    rg16 = lax.shift_right_logical(rows, 3)
    rs16 = lax.bitwise_and(rows, 7)
    two = jnp.full((16,), 2, jnp.int32)
    for c in range(D_POS):
      vals = plsc.load_gather(p1_v, [hi + c])
      plsc.store_scatter(obuf_v, [rg16, two, rs16, jnp.full((16,), D_WORD - 256 + c, jnp.int32)], vals)
    for c in range(D_POS):
      vals = plsc.load_gather(p2_v, [ti + c])
      plsc.store_scatter(obuf_v, [rg16, two, rs16, jnp.full((16,), D_WORD - 256 + D_POS + c, jnp.int32)], vals)


def _body(word_hbm, posh_hbm, post_hbm, wtab_hbm, p1_hbm, p2_hbm, out_hbm,
          widx_v, hidx_v, tidx_v, p1_v, p2_v,
          wbuf0, wbuf1, obuf0, obuf1, sem_g0, sem_g1, sem_w0, sem_w1):
  wid = lax.axis_index("s") * NC + lax.axis_index("c")
  crow = wid * CHUNKS

  pltpu.sync_copy(word_hbm.at[wid], widx_v)
  pltpu.sync_copy(posh_hbm.at[wid], hidx_v)
  pltpu.sync_copy(post_hbm.at[wid], tidx_v)
  pltpu.sync_copy(p1_hbm, p1_v)
  pltpu.sync_copy(p2_hbm, p2_v)

  iota = lax.iota(jnp.int32, 16)

  def gather(i, wbuf, sem):
    return pltpu.async_copy(wtab_hbm.at[widx_v.at[i]], wbuf, sem)

  def writeback(i, obuf, sem):
    return pltpu.async_copy(obuf, out_hbm.at[pl.ds((crow + i) * (CH // 8), CH // 8)], sem)

  gather(0, wbuf0, sem_g0)

  @pl.loop(0, CHUNKS, step=2)
  def pair(i):
    gather(i + 1, wbuf1, sem_g1)
    pltpu.make_async_copy(wtab_hbm.at[widx_v.at[i]], wbuf0, sem_g0).wait()

    @pl.when(i > 0)
    def _():
      pltpu.make_async_copy(obuf0, out_hbm.at[pl.ds((crow + i - 2) * (CH // 8), CH // 8)], sem_w0).wait()

    _assemble(i, wbuf0, obuf0, hidx_v, tidx_v, p1_v, p2_v, iota)
    writeback(i, obuf0, sem_w0)

    @pl.when(i + 2 < CHUNKS)
    def _():
      gather(i + 2, wbuf0, sem_g0)
    pltpu.make_async_copy(wtab_hbm.at[widx_v.at[i + 1]], wbuf1, sem_g1).wait()

    @pl.when(i > 0)
    def _():
      pltpu.make_async_copy(obuf1, out_hbm.at[pl.ds((crow + i - 1) * (CH // 8), CH // 8)], sem_w1).wait()

    _assemble(i + 1, wbuf1, obuf1, hidx_v, tidx_v, p1_v, p2_v, iota)
    writeback(i + 1, obuf1, sem_w1)

  pltpu.make_async_copy(obuf0, out_hbm.at[pl.ds((crow + CHUNKS - 2) * (CH // 8), CH // 8)], sem_w0).wait()
  pltpu.make_async_copy(obuf1, out_hbm.at[pl.ds((crow + CHUNKS - 1) * (CH // 8), CH // 8)], sem_w1).wait()


V_PAD = 100096
CTILES = D_PAD // 8
WTILES = V_PAD // 128


def _tbody(wt4_hbm, out_hbm, slab0, slab1, rowA, rowB, sem_i0, sem_i1, sem_wA, sem_wB):
  wid = lax.axis_index("s") * NC + lax.axis_index("c")
  iota = lax.iota(jnp.int32, 16)

  def stage(j, slab, sem):
    return pltpu.async_copy(wt4_hbm.at[:, pl.ds(j * NW + wid, 1)], slab, sem)

  def halfstep(j, slab, sem):
    g = j * NW + wid

    @pl.when(g < WTILES)
    def _():
      pltpu.make_async_copy(wt4_hbm.at[:, pl.ds(g, 1)], slab, sem).wait()

      for half, row_v, sem_w in ((0, rowA, sem_wA), (1, rowB, sem_wB)):
        @pl.when(j > 0)
        def _():
          pltpu.make_async_copy(
              row_v, out_hbm.at[pl.ds((g - NW) * 128 + half * 64, 64)], sem_w).wait()

        @pl.loop(0, CTILES)
        def ca(a):
          for cs in range(8):
            col = jnp.full((16,), 8 * a + cs, jnp.int32)
            for wg in range(4):
              vals = slab[a, 0, cs, pl.ds((half * 4 + wg) * 16, 16)]
              plsc.store_scatter(row_v, [iota + wg * 16, col], vals)

        pltpu.async_copy(row_v, out_hbm.at[pl.ds(g * 128 + half * 64, 64)], sem_w)

  stage(0, slab0, sem_i0)

  @pl.loop(0, 26, step=2)
  def pair(j):
    @pl.when((j + 1) * NW + wid < WTILES)
    def _():
      stage(j + 1, slab1, sem_i1)
    halfstep(j, slab0, sem_i0)

    @pl.when((j + 2) * NW + wid < WTILES)
    def _():
      stage(j + 2, slab0, sem_i0)
    halfstep(j + 1, slab1, sem_i1)

  glast = jnp.where(wid < WTILES - 24 * NW, 24 * NW + wid, 23 * NW + wid)
  pltpu.make_async_copy(rowA, out_hbm.at[pl.ds(glast * 128, 64)], sem_wA).wait()
  pltpu.make_async_copy(rowB, out_hbm.at[pl.ds(glast * 128 + 64, 64)], sem_wB).wait()


_tmesh = plsc.VectorSubcoreMesh(core_axis_name="c", subcore_axis_name="s")

_transpose_call = pl.kernel(
    _tbody,
    out_type=jax.ShapeDtypeStruct((V_PAD, D_PAD), jnp.float32),
    mesh=_tmesh,
    scratch_types=[
        pltpu.VMEM((CTILES, 1, 8, 128), jnp.float32),
        pltpu.VMEM((CTILES, 1, 8, 128), jnp.float32),
        pltpu.VMEM((64, D_PAD), jnp.float32),
        pltpu.VMEM((64, D_PAD), jnp.float32),
        pltpu.SemaphoreType.DMA,
        pltpu.SemaphoreType.DMA,
        pltpu.SemaphoreType.DMA,
        pltpu.SemaphoreType.DMA,
    ],
    compiler_params=pltpu.CompilerParams(
        use_tc_tiling_on_sc=False, needs_layout_passes=False),
)


_mesh = plsc.VectorSubcoreMesh(core_axis_name="c", subcore_axis_name="s")

_sc_call = pl.kernel(
    _body,
    out_type=jax.ShapeDtypeStruct((N // 8, 3, 8, 128), jnp.float32),
    mesh=_mesh,
    scratch_types=[
        pltpu.VMEM((CHUNKS, CH), jnp.int32),
        pltpu.VMEM((CHUNKS, CH), jnp.int32),
        pltpu.VMEM((CHUNKS, CH), jnp.int32),
        pltpu.VMEM((NPOS * D_POS,), jnp.float32),
        pltpu.VMEM((NPOS * D_POS,), jnp.float32),
        pltpu.VMEM((CH, D_PAD), jnp.float32),
        pltpu.VMEM((CH, D_PAD), jnp.float32),
        pltpu.VMEM((CH // 8, 3, 8, 128), jnp.float32),
        pltpu.VMEM((CH // 8, 3, 8, 128), jnp.float32),
        pltpu.SemaphoreType.DMA,
        pltpu.SemaphoreType.DMA,
        pltpu.SemaphoreType.DMA,
        pltpu.SemaphoreType.DMA,
    ],
    compiler_params=pltpu.CompilerParams(
        use_tc_tiling_on_sc=False, needs_layout_passes=False),
)


@jax.jit
def kernel(word, posh, post, word_table, pos1_table, pos2_table):
  w = word.reshape(NW, CHUNKS, CH).astype(jnp.int32)
  h = posh.reshape(NW, CHUNKS, CH).astype(jnp.int32)
  t = post.reshape(NW, CHUNKS, CH).astype(jnp.int32)
  p1 = pos1_table.reshape(NPOS * D_POS)
  p2 = pos2_table.reshape(NPOS * D_POS)
  wtP = jnp.pad(word_table, ((0, V_PAD - word_table.shape[0]), (0, D_PAD - D_WORD)))
  wt4 = wtP.T.reshape(CTILES, 8, WTILES, 128).transpose(0, 2, 1, 3)
  wtRM = _transpose_call(wt4)
  out4 = _sc_call(w, h, t, wtRM, p1, p2)
  out = out4.transpose(0, 2, 1, 3).reshape(N, 384)[:, :D_OUT]
  return out.reshape(B, L, D_OUT)


# speedup vs baseline: 1.5920x; 1.0007x over previous
import functools
import jax
import jax.numpy as jnp
from jax import lax
from jax.experimental import pallas as pl
from jax.experimental.pallas import tpu as pltpu
from jax.experimental.pallas import tpu_sc as plsc

B = 1024
L = 200
D_WORD = 300
D_POS = 5
D_OUT = D_WORD + 2 * D_POS
D_PAD = 304
NPOS = 400

N = B * L
CH = 64
NC = 2
NS = 16
NW = NC * NS
N_PER_W = N // NW
CHUNKS = N_PER_W // CH
G = 16


def _assemble(i, wbuf_v, obuf_v, hidx_v, tidx_v, p1_v, p2_v, iota):
  @pl.loop(0, CH // G)
  def group(g):
    r0 = g * G
    rows = iota + r0
    for rr in range(G):
      r = r0 + rr
      rg = r // 8
      rs = r % 8
      for k in range(D_PAD // G):
        obuf_v[rg, k // 8, rs, pl.ds((k % 8) * G, G)] = wbuf_v[r, pl.ds(k * G, G)]
    hi = hidx_v[i, pl.ds(r0, G)] * D_POS
    ti = tidx_v[i, pl.ds(r0, G)] * D_POS
    rg16 = lax.shift_right_logical(rows, 3)
    rs16 = lax.bitwise_and(rows, 7)
    two = jnp.full((16,), 2, jnp.int32)
    for c in range(D_POS):
      vals = plsc.load_gather(p1_v, [hi + c])
      plsc.store_scatter(obuf_v, [rg16, two, rs16, jnp.full((16,), D_WORD - 256 + c, jnp.int32)], vals)
    for c in range(D_POS):
      vals = plsc.load_gather(p2_v, [ti + c])
      plsc.store_scatter(obuf_v, [rg16, two, rs16, jnp.full((16,), D_WORD - 256 + D_POS + c, jnp.int32)], vals)


def _body(word_hbm, posh_hbm, post_hbm, wtab_hbm, p1_hbm, p2_hbm, out_hbm,
          widx_v, hidx_v, tidx_v, p1_v, p2_v,
          wbuf0, wbuf1, obuf0, obuf1, sem_g0, sem_g1, sem_w0, sem_w1,
          sem_g0b, sem_g1b):
  wid = lax.axis_index("s") * NC + lax.axis_index("c")
  crow = wid * CHUNKS

  pltpu.sync_copy(word_hbm.at[wid], widx_v)
  pltpu.sync_copy(posh_hbm.at[wid], hidx_v)
  pltpu.sync_copy(post_hbm.at[wid], tidx_v)
  pltpu.sync_copy(p1_hbm, p1_v)
  pltpu.sync_copy(p2_hbm, p2_v)

  iota = lax.iota(jnp.int32, 16)

  def gather(i, wbuf, sem, semb):
    pltpu.async_copy(wtab_hbm.at[widx_v.at[i, pl.ds(0, CH // 2)]],
                     wbuf.at[pl.ds(0, CH // 2)], sem)
    pltpu.async_copy(wtab_hbm.at[widx_v.at[i, pl.ds(CH // 2, CH // 2)]],
                     wbuf.at[pl.ds(CH // 2, CH // 2)], semb)

  def gather_wait(i, wbuf, sem, semb):
    pltpu.make_async_copy(wtab_hbm.at[widx_v.at[i, pl.ds(0, CH // 2)]],
                          wbuf.at[pl.ds(0, CH // 2)], sem).wait()
    pltpu.make_async_copy(wtab_hbm.at[widx_v.at[i, pl.ds(CH // 2, CH // 2)]],
                          wbuf.at[pl.ds(CH // 2, CH // 2)], semb).wait()

  def writeback(i, obuf, sem):
    return pltpu.async_copy(obuf, out_hbm.at[pl.ds((crow + i) * (CH // 8), CH // 8)], sem)

  gather(0, wbuf0, sem_g0, sem_g0b)

  @pl.loop(0, CHUNKS, step=2)
  def pair(i):
    gather(i + 1, wbuf1, sem_g1, sem_g1b)
    gather_wait(i, wbuf0, sem_g0, sem_g0b)

    @pl.when(i > 0)
    def _():
      pltpu.make_async_copy(obuf0, out_hbm.at[pl.ds((crow + i - 2) * (CH // 8), CH // 8)], sem_w0).wait()

    _assemble(i, wbuf0, obuf0, hidx_v, tidx_v, p1_v, p2_v, iota)
    writeback(i, obuf0, sem_w0)

    @pl.when(i + 2 < CHUNKS)
    def _():
      gather(i + 2, wbuf0, sem_g0, sem_g0b)
    gather_wait(i + 1, wbuf1, sem_g1, sem_g1b)

    @pl.when(i > 0)
    def _():
      pltpu.make_async_copy(obuf1, out_hbm.at[pl.ds((crow + i - 1) * (CH // 8), CH // 8)], sem_w1).wait()

    _assemble(i + 1, wbuf1, obuf1, hidx_v, tidx_v, p1_v, p2_v, iota)
    writeback(i + 1, obuf1, sem_w1)

  pltpu.make_async_copy(obuf0, out_hbm.at[pl.ds((crow + CHUNKS - 2) * (CH // 8), CH // 8)], sem_w0).wait()
  pltpu.make_async_copy(obuf1, out_hbm.at[pl.ds((crow + CHUNKS - 1) * (CH // 8), CH // 8)], sem_w1).wait()


V_PAD = 100096
CTILES = D_PAD // 8
WTILES = V_PAD // 128


def _tbody(wt4_hbm, out_hbm, slab0, slab1, rowA, rowB, sem_i0, sem_i1, sem_wA, sem_wB):
  wid = lax.axis_index("s") * NC + lax.axis_index("c")
  iota = lax.iota(jnp.int32, 16)

  def stage(j, slab, sem):
    return pltpu.async_copy(wt4_hbm.at[:, pl.ds(j * NW + wid, 1)], slab, sem)

  def halfstep(j, slab, sem):
    g = j * NW + wid

    @pl.when(g < WTILES)
    def _():
      pltpu.make_async_copy(wt4_hbm.at[:, pl.ds(g, 1)], slab, sem).wait()

      for half, row_v, sem_w in ((0, rowA, sem_wA), (1, rowB, sem_wB)):
        @pl.when(j > 0)
        def _():
          pltpu.make_async_copy(
              row_v, out_hbm.at[pl.ds((g - NW) * 128 + half * 64, 64)], sem_w).wait()

        @pl.loop(0, CTILES)
        def ca(a):
          for cs in range(8):
            col = jnp.full((16,), 8 * a + cs, jnp.int32)
            for wg in range(4):
              vals = slab[a, 0, cs, pl.ds((half * 4 + wg) * 16, 16)]
              plsc.store_scatter(row_v, [iota + wg * 16, col], vals)

        pltpu.async_copy(row_v, out_hbm.at[pl.ds(g * 128 + half * 64, 64)], sem_w)

  stage(0, slab0, sem_i0)

  @pl.loop(0, 26, step=2)
  def pair(j):
    @pl.when((j + 1) * NW + wid < WTILES)
    def _():
      stage(j + 1, slab1, sem_i1)
    halfstep(j, slab0, sem_i0)

    @pl.when((j + 2) * NW + wid < WTILES)
    def _():
      stage(j + 2, slab0, sem_i0)
    halfstep(j + 1, slab1, sem_i1)

  glast = jnp.where(wid < WTILES - 24 * NW, 24 * NW + wid, 23 * NW + wid)
  pltpu.make_async_copy(rowA, out_hbm.at[pl.ds(glast * 128, 64)], sem_wA).wait()
  pltpu.make_async_copy(rowB, out_hbm.at[pl.ds(glast * 128 + 64, 64)], sem_wB).wait()


_tmesh = plsc.VectorSubcoreMesh(core_axis_name="c", subcore_axis_name="s")

_transpose_call = pl.kernel(
    _tbody,
    out_type=jax.ShapeDtypeStruct((V_PAD, D_PAD), jnp.float32),
    mesh=_tmesh,
    scratch_types=[
        pltpu.VMEM((CTILES, 1, 8, 128), jnp.float32),
        pltpu.VMEM((CTILES, 1, 8, 128), jnp.float32),
        pltpu.VMEM((64, D_PAD), jnp.float32),
        pltpu.VMEM((64, D_PAD), jnp.float32),
        pltpu.SemaphoreType.DMA,
        pltpu.SemaphoreType.DMA,
        pltpu.SemaphoreType.DMA,
        pltpu.SemaphoreType.DMA,
    ],
    compiler_params=pltpu.CompilerParams(
        use_tc_tiling_on_sc=False, needs_layout_passes=False),
)


_mesh = plsc.VectorSubcoreMesh(core_axis_name="c", subcore_axis_name="s")

_sc_call = pl.kernel(
    _body,
    out_type=jax.ShapeDtypeStruct((N // 8, 3, 8, 128), jnp.float32),
    mesh=_mesh,
    scratch_types=[
        pltpu.VMEM((CHUNKS, CH), jnp.int32),
        pltpu.VMEM((CHUNKS, CH), jnp.int32),
        pltpu.VMEM((CHUNKS, CH), jnp.int32),
        pltpu.VMEM((NPOS * D_POS,), jnp.float32),
        pltpu.VMEM((NPOS * D_POS,), jnp.float32),
        pltpu.VMEM((CH, D_PAD), jnp.float32),
        pltpu.VMEM((CH, D_PAD), jnp.float32),
        pltpu.VMEM((CH // 8, 3, 8, 128), jnp.float32),
        pltpu.VMEM((CH // 8, 3, 8, 128), jnp.float32),
        pltpu.SemaphoreType.DMA,
        pltpu.SemaphoreType.DMA,
        pltpu.SemaphoreType.DMA,
        pltpu.SemaphoreType.DMA,
        pltpu.SemaphoreType.DMA,
        pltpu.SemaphoreType.DMA,
    ],
    compiler_params=pltpu.CompilerParams(
        use_tc_tiling_on_sc=False, needs_layout_passes=False),
)


@jax.jit
def kernel(word, posh, post, word_table, pos1_table, pos2_table):
  w = word.reshape(NW, CHUNKS, CH).astype(jnp.int32)
  h = posh.reshape(NW, CHUNKS, CH).astype(jnp.int32)
  t = post.reshape(NW, CHUNKS, CH).astype(jnp.int32)
  p1 = pos1_table.reshape(NPOS * D_POS)
  p2 = pos2_table.reshape(NPOS * D_POS)
  wtP = jnp.pad(word_table, ((0, V_PAD - word_table.shape[0]), (0, D_PAD - D_WORD)))
  wt4 = wtP.T.reshape(CTILES, 8, WTILES, 128).transpose(0, 2, 1, 3)
  wtRM = _transpose_call(wt4)
  out4 = _sc_call(w, h, t, wtRM, p1, p2)
  out = out4.transpose(0, 2, 1, 3).reshape(N, 384)[:, :D_OUT]
  return out.reshape(B, L, D_OUT)


# speedup vs baseline: 1.7046x; 1.0707x over previous
import functools
import jax
import jax.numpy as jnp
from jax import lax
from jax.experimental import pallas as pl
from jax.experimental.pallas import tpu as pltpu
from jax.experimental.pallas import tpu_sc as plsc

B = 1024
L = 200
D_WORD = 300
D_POS = 5
D_OUT = D_WORD + 2 * D_POS
D_PAD = 304
NPOS = 400

N = B * L
CH = 64
NC = 2
NS = 16
NW = NC * NS
N_PER_W = N // NW
CHUNKS = N_PER_W // CH
G = 16


def _assemble(i, wbuf_v, obuf_v, hidx_v, tidx_v, p1_v, p2_v, iota):
  @pl.loop(0, CH // G)
  def group(g):
    r0 = g * G
    rows = iota + r0
    for rr in range(G):
      r = r0 + rr
      rg = r // 8
      rs = r % 8
      for k in range(D_PAD // G):
        obuf_v[rg, k // 8, rs, pl.ds((k % 8) * G, G)] = wbuf_v[r, pl.ds(k * G, G)]
    hi = hidx_v[i, pl.ds(r0, G)] * D_POS
    ti = tidx_v[i, pl.ds(r0, G)] * D_POS
    rg16 = lax.shift_right_logical(rows, 3)
    rs16 = lax.bitwise_and(rows, 7)
    two = jnp.full((16,), 2, jnp.int32)
    for c in range(D_POS):
      vals = plsc.load_gather(p1_v, [hi + c])
      plsc.store_scatter(obuf_v, [rg16, two, rs16, jnp.full((16,), D_WORD - 256 + c, jnp.int32)], vals)
    for c in range(D_POS):
      vals = plsc.load_gather(p2_v, [ti + c])
      plsc.store_scatter(obuf_v, [rg16, two, rs16, jnp.full((16,), D_WORD - 256 + D_POS + c, jnp.int32)], vals)


def _body(word_hbm, posh_hbm, post_hbm, wtab_hbm, p1_hbm, p2_hbm, out_hbm,
          widx_v, hidx_v, tidx_v, p1_v, p2_v,
          wbuf0, wbuf1, obuf0, obuf1, sem_g0, sem_g1, sem_w0, sem_w1,
          sem_g0b, sem_g1b):
  wid = lax.axis_index("s") * NC + lax.axis_index("c")
  crow = wid * CHUNKS

  pltpu.sync_copy(word_hbm.at[wid], widx_v)
  pltpu.sync_copy(posh_hbm.at[wid], hidx_v)
  pltpu.sync_copy(post_hbm.at[wid], tidx_v)
  pltpu.sync_copy(p1_hbm, p1_v)
  pltpu.sync_copy(p2_hbm, p2_v)

  iota = lax.iota(jnp.int32, 16)

  def gather(i, wbuf, sem, semb):
    pltpu.async_copy(wtab_hbm.at[widx_v.at[i, pl.ds(0, CH // 2)]],
                     wbuf.at[pl.ds(0, CH // 2)], sem)
    pltpu.async_copy(wtab_hbm.at[widx_v.at[i, pl.ds(CH // 2, CH // 2)]],
                     wbuf.at[pl.ds(CH // 2, CH // 2)], semb)

  def gather_wait(i, wbuf, sem, semb):
    pltpu.make_async_copy(wtab_hbm.at[widx_v.at[i, pl.ds(0, CH // 2)]],
                          wbuf.at[pl.ds(0, CH // 2)], sem).wait()
    pltpu.make_async_copy(wtab_hbm.at[widx_v.at[i, pl.ds(CH // 2, CH // 2)]],
                          wbuf.at[pl.ds(CH // 2, CH // 2)], semb).wait()

  def writeback(i, obuf, sem):
    return pltpu.async_copy(obuf, out_hbm.at[pl.ds((crow + i) * (CH // 8), CH // 8)], sem)

  gather(0, wbuf0, sem_g0, sem_g0b)

  @pl.loop(0, CHUNKS, step=2)
  def pair(i):
    gather(i + 1, wbuf1, sem_g1, sem_g1b)
    gather_wait(i, wbuf0, sem_g0, sem_g0b)

    @pl.when(i > 0)
    def _():
      pltpu.make_async_copy(obuf0, out_hbm.at[pl.ds((crow + i - 2) * (CH // 8), CH // 8)], sem_w0).wait()

    _assemble(i, wbuf0, obuf0, hidx_v, tidx_v, p1_v, p2_v, iota)
    writeback(i, obuf0, sem_w0)

    @pl.when(i + 2 < CHUNKS)
    def _():
      gather(i + 2, wbuf0, sem_g0, sem_g0b)
    gather_wait(i + 1, wbuf1, sem_g1, sem_g1b)

    @pl.when(i > 0)
    def _():
      pltpu.make_async_copy(obuf1, out_hbm.at[pl.ds((crow + i - 1) * (CH // 8), CH // 8)], sem_w1).wait()

    _assemble(i + 1, wbuf1, obuf1, hidx_v, tidx_v, p1_v, p2_v, iota)
    writeback(i + 1, obuf1, sem_w1)

  pltpu.make_async_copy(obuf0, out_hbm.at[pl.ds((crow + CHUNKS - 2) * (CH // 8), CH // 8)], sem_w0).wait()
  pltpu.make_async_copy(obuf1, out_hbm.at[pl.ds((crow + CHUNKS - 1) * (CH // 8), CH // 8)], sem_w1).wait()


V_PAD = 100096
CTILES = D_PAD // 8
WTILES = V_PAD // 128


def _tbody(wt4_hbm, out_hbm, slab0, slab1, rowA, rowB, sem_i0, sem_i1, sem_wA, sem_wB):
  wid = lax.axis_index("s") * NC + lax.axis_index("c")
  iota = lax.iota(jnp.int32, 16)

  def stage(j, slab, sem):
    return pltpu.async_copy(wt4_hbm.at[:, pl.ds(j * NW + wid, 1)], slab, sem)

  def halfstep(j, slab, sem):
    g = j * NW + wid

    @pl.when(g < WTILES)
    def _():
      pltpu.make_async_copy(wt4_hbm.at[:, pl.ds(g, 1)], slab, sem).wait()

      for half, row_v, sem_w in ((0, rowA, sem_wA), (1, rowB, sem_wB)):
        @pl.when(j > 0)
        def _():
          pltpu.make_async_copy(
              row_v.at[:, pl.ds(0, D_PAD)],
              out_hbm.at[pl.ds((g - NW) * 128 + half * 64, 64)], sem_w).wait()

        @pl.loop(0, CTILES)
        def ca(a):
          for cs in range(8):
            col = jnp.full((16,), 8 * a + cs, jnp.int32)
            for wg in range(4):
              vals = slab[a, 0, cs, pl.ds((half * 4 + wg) * 16, 16)]
              plsc.store_scatter(row_v, [iota + wg * 16, col], vals)

        pltpu.async_copy(row_v.at[:, pl.ds(0, D_PAD)],
                         out_hbm.at[pl.ds(g * 128 + half * 64, 64)], sem_w)

  stage(0, slab0, sem_i0)

  @pl.loop(0, 26, step=2)
  def pair(j):
    @pl.when((j + 1) * NW + wid < WTILES)
    def _():
      stage(j + 1, slab1, sem_i1)
    halfstep(j, slab0, sem_i0)

    @pl.when((j + 2) * NW + wid < WTILES)
    def _():
      stage(j + 2, slab0, sem_i0)
    halfstep(j + 1, slab1, sem_i1)

  glast = jnp.where(wid < WTILES - 24 * NW, 24 * NW + wid, 23 * NW + wid)
  pltpu.make_async_copy(rowA.at[:, pl.ds(0, D_PAD)],
                        out_hbm.at[pl.ds(glast * 128, 64)], sem_wA).wait()
  pltpu.make_async_copy(rowB.at[:, pl.ds(0, D_PAD)],
                        out_hbm.at[pl.ds(glast * 128 + 64, 64)], sem_wB).wait()


_tmesh = plsc.VectorSubcoreMesh(core_axis_name="c", subcore_axis_name="s")

_transpose_call = pl.kernel(
    _tbody,
    out_type=jax.ShapeDtypeStruct((V_PAD, D_PAD), jnp.float32),
    mesh=_tmesh,
    scratch_types=[
        pltpu.VMEM((CTILES, 1, 8, 128), jnp.float32),
        pltpu.VMEM((CTILES, 1, 8, 128), jnp.float32),
        pltpu.VMEM((64, 322), jnp.float32),
        pltpu.VMEM((64, 322), jnp.float32),
        pltpu.SemaphoreType.DMA,
        pltpu.SemaphoreType.DMA,
        pltpu.SemaphoreType.DMA,
        pltpu.SemaphoreType.DMA,
    ],
    compiler_params=pltpu.CompilerParams(
        use_tc_tiling_on_sc=False, needs_layout_passes=False),
)


_mesh = plsc.VectorSubcoreMesh(core_axis_name="c", subcore_axis_name="s")

_sc_call = pl.kernel(
    _body,
    out_type=jax.ShapeDtypeStruct((N // 8, 3, 8, 128), jnp.float32),
    mesh=_mesh,
    scratch_types=[
        pltpu.VMEM((CHUNKS, CH), jnp.int32),
        pltpu.VMEM((CHUNKS, CH), jnp.int32),
        pltpu.VMEM((CHUNKS, CH), jnp.int32),
        pltpu.VMEM((NPOS * D_POS,), jnp.float32),
        pltpu.VMEM((NPOS * D_POS,), jnp.float32),
        pltpu.VMEM((CH, D_PAD), jnp.float32),
        pltpu.VMEM((CH, D_PAD), jnp.float32),
        pltpu.VMEM((CH // 8, 3, 8, 128), jnp.float32),
        pltpu.VMEM((CH // 8, 3, 8, 128), jnp.float32),
        pltpu.SemaphoreType.DMA,
        pltpu.SemaphoreType.DMA,
        pltpu.SemaphoreType.DMA,
        pltpu.SemaphoreType.DMA,
        pltpu.SemaphoreType.DMA,
        pltpu.SemaphoreType.DMA,
    ],
    compiler_params=pltpu.CompilerParams(
        use_tc_tiling_on_sc=False, needs_layout_passes=False),
)


@jax.jit
def kernel(word, posh, post, word_table, pos1_table, pos2_table):
  w = word.reshape(NW, CHUNKS, CH).astype(jnp.int32)
  h = posh.reshape(NW, CHUNKS, CH).astype(jnp.int32)
  t = post.reshape(NW, CHUNKS, CH).astype(jnp.int32)
  p1 = pos1_table.reshape(NPOS * D_POS)
  p2 = pos2_table.reshape(NPOS * D_POS)
  wtP = jnp.pad(word_table, ((0, V_PAD - word_table.shape[0]), (0, D_PAD - D_WORD)))
  wt4 = wtP.T.reshape(CTILES, 8, WTILES, 128).transpose(0, 2, 1, 3)
  wtRM = _transpose_call(wt4)
  out4 = _sc_call(w, h, t, wtRM, p1, p2)
  out = out4.transpose(0, 2, 1, 3).reshape(N, 384)[:, :D_OUT]
  return out.reshape(B, L, D_OUT)
